# baseline (speedup 1.0000x reference)
import functools

import jax
import jax.numpy as jnp
from jax import lax
from jax.experimental import pallas as pl
from jax.experimental.pallas import tpu as pltpu
from jax.experimental.pallas import tpu_sc as plsc

N = 10000
D_IN = 128
DH = 16
DO = 8
NP = 10240
NZ = NP - N
E_RAW = 320000
E_TOT = E_RAW + N
NC, NS, L = 2, 16, 16
CHUNK = 128
NBUF = 4
CH0 = 60
CH1 = 104
CHMAX = max(CH0, CH1)
TOTAL_CH = NS * (CH0 + CH1)
ROWS_PAD = TOTAL_CH + CHMAX - min(CH0, CH1)
E_PAD = ROWS_PAD * CHUNK
RPS = NP // NS

_mesh = functools.partial(
    plsc.VectorSubcoreMesh, core_axis_name="c", subcore_axis_name="s"
)
_sc_params = pltpu.CompilerParams(use_tc_tiling_on_sc=False)


def _tile_plan(c, s):
    start = jnp.where(c == 0, s * CH0, NS * CH0 + s * CH1)
    nch = jnp.where(c == 0, CH0, CH1)
    ngrp = jnp.where(c == 0, CH0 // NBUF, CH1 // NBUF)
    return start, nch, ngrp


@functools.partial(
    pl.kernel,
    out_type=jax.ShapeDtypeStruct((NC, NP), jnp.float32),
    mesh=_mesh(),
    scratch_types=[
        pltpu.VMEM_SHARED((NP,), jnp.float32),
        pltpu.VMEM((CHMAX, CHUNK), jnp.int32),
        pltpu.VMEM((CHUNK,), jnp.float32),
        pltpu.VMEM((RPS,), jnp.float32),
        pltpu.SemaphoreType.DMA((NBUF,)),
        pltpu.SemaphoreType.DMA,
    ],
    compiler_params=_sc_params,
)
def _deg_kernel(dst_hbm, out_hbm, acc_sh, dst_v, ones_v, zero_v, ssem, isem):
    c = lax.axis_index("c")
    s = lax.axis_index("s")
    start, nch, ngrp = _tile_plan(c, s)

    idx_cp = pltpu.async_copy(dst_hbm.at[pl.ds(start, CHMAX)], dst_v, isem)

    def fill_ones(i, carry):
        ones_v[pl.ds(i * L, L)] = jnp.full((L,), 1.0, jnp.float32)
        return carry

    lax.fori_loop(0, CHUNK // L, fill_ones, 0)

    def fill_zero(i, carry):
        zero_v[pl.ds(i * L, L)] = jnp.zeros((L,), jnp.float32)
        return carry

    lax.fori_loop(0, RPS // L, fill_zero, 0)

    pltpu.sync_copy(zero_v, acc_sh.at[pl.ds(s * RPS, RPS)])
    idx_cp.wait()
    plsc.subcore_barrier()

    def group(g, carry):
        for b in range(NBUF):
            j = g * NBUF + b
            pltpu.async_copy(ones_v, acc_sh.at[dst_v.at[j]], ssem.at[b],
                             add=True)
        for b in range(NBUF):
            pltpu.make_async_copy(ones_v, acc_sh.at[dst_v.at[b]],
                                  ssem.at[b]).wait()
        return carry

    lax.fori_loop(0, ngrp, group, 0)

    plsc.subcore_barrier()
    pltpu.sync_copy(
        acc_sh.at[pl.ds(s * RPS, RPS)], out_hbm.at[c, pl.ds(s * RPS, RPS)]
    )


def _agg_body(D, feat_hbm, src_hbm, dst_hbm, out_hbm,
              acc_sh, src_v, dst_v, rows_v, gsem, ssem, isem):
    c = lax.axis_index("c")
    s = lax.axis_index("s")
    start, nch, ngrp = _tile_plan(c, s)

    cp_s = pltpu.async_copy(src_hbm.at[pl.ds(start, CHMAX)], src_v, isem)
    cp_d = pltpu.async_copy(dst_hbm.at[pl.ds(start, CHMAX)], dst_v, isem)

    base = s * RPS
    pltpu.sync_copy(feat_hbm.at[pl.ds(N, NZ)], acc_sh.at[pl.ds(base, NZ)])
    pltpu.sync_copy(feat_hbm.at[pl.ds(N, NZ)],
                    acc_sh.at[pl.ds(base + NZ, NZ)])
    pltpu.sync_copy(feat_hbm.at[pl.ds(N, RPS - 2 * NZ)],
                    acc_sh.at[pl.ds(base + 2 * NZ, RPS - 2 * NZ)])
    cp_s.wait()
    cp_d.wait()
    plsc.subcore_barrier()

    for b in range(NBUF):
        pltpu.async_copy(feat_hbm.at[src_v.at[b]], rows_v.at[b], gsem.at[b])

    def group(g, carry):
        for b in range(NBUF):
            j = g * NBUF + b
            pltpu.make_async_copy(feat_hbm.at[src_v.at[b]], rows_v.at[b],
                                  gsem.at[b]).wait()
            pltpu.async_copy(rows_v.at[b], acc_sh.at[dst_v.at[j]],
                             ssem.at[b], add=True)
        for b in range(NBUF):
            nxt = g * NBUF + b + NBUF
            pltpu.make_async_copy(rows_v.at[b], acc_sh.at[dst_v.at[b]],
                                  ssem.at[b]).wait()

            @pl.when(nxt < nch)
            def _():
                pltpu.async_copy(feat_hbm.at[src_v.at[nxt]], rows_v.at[b],
                                 gsem.at[b])

        return carry

    lax.fori_loop(0, ngrp, group, 0)

    plsc.subcore_barrier()
    pltpu.sync_copy(
        acc_sh.at[pl.ds(s * RPS, RPS)], out_hbm.at[c, pl.ds(s * RPS, RPS)]
    )


def _make_agg(D):
    return functools.partial(
        pl.kernel,
        out_type=jax.ShapeDtypeStruct((NC, NP, D), jnp.float32),
        mesh=_mesh(),
        scratch_types=[
            pltpu.VMEM_SHARED((NP, D), jnp.float32),
            pltpu.VMEM((CHMAX, CHUNK), jnp.int32),
            pltpu.VMEM((CHMAX, CHUNK), jnp.int32),
            pltpu.VMEM((NBUF, CHUNK, D), jnp.float32),
            pltpu.SemaphoreType.DMA((NBUF,)),
            pltpu.SemaphoreType.DMA((NBUF,)),
            pltpu.SemaphoreType.DMA,
        ],
        compiler_params=_sc_params,
    )(functools.partial(_agg_body, D))


_agg16 = _make_agg(DH)
_agg4 = _make_agg(DO)


def _tca_body(x_ref, w1_ref, deg_ref, hs_ref, dis_ref):
    deg = deg_ref[0] + deg_ref[1]
    dis = jnp.where(deg > 0, lax.rsqrt(jnp.maximum(deg, 1e-12)), 0.0)
    dis_ref[...] = dis
    h = jnp.dot(x_ref[...], w1_ref[...], preferred_element_type=jnp.float32)
    h = jnp.concatenate([h, jnp.zeros((NP - N, DH), jnp.float32)], axis=0)
    hs_ref[...] = h * dis[:, None]


def _tcb_body(s1_ref, dis_ref, b1_ref, w2_ref, gs_ref):
    s = s1_ref[0] + s1_ref[1]
    dis = dis_ref[...]
    x2 = jnp.maximum(s * dis[:, None] + b1_ref[...][None, :], 0.0)
    g = jnp.dot(x2, w2_ref[...], preferred_element_type=jnp.float32)
    g = g * dis[:, None]
    row = lax.broadcasted_iota(jnp.int32, (NP, DO), 0)
    gs_ref[...] = jnp.where(row < N, g, 0.0)


def _tcc_body(s2_ref, dis_ref, b2_ref, out_ref):
    su = s2_ref[0] + s2_ref[1]
    dis = dis_ref[...]
    z = su[:, :2] * dis[:, None] + b2_ref[...][None, :]
    m = jnp.max(z, axis=1, keepdims=True)
    lse = m + jnp.log(jnp.sum(jnp.exp(z - m), axis=1, keepdims=True))
    out_ref[...] = z - lse


def kernel(x, edge_index, W1, b1, W2, b2):
    loop = jnp.arange(N, dtype=jnp.int32)
    pad = jnp.full((E_PAD - E_TOT,), N, jnp.int32)
    src = jnp.concatenate([edge_index[0], loop, pad]).reshape(ROWS_PAD, CHUNK)
    dst = jnp.concatenate([edge_index[1], loop, pad]).reshape(ROWS_PAD, CHUNK)
    W2p = jnp.pad(W2, ((0, 0), (0, DO - 2)))

    deg2 = _deg_kernel(dst)

    hs, dis = pl.pallas_call(
        _tca_body,
        out_shape=(
            jax.ShapeDtypeStruct((NP, DH), jnp.float32),
            jax.ShapeDtypeStruct((NP,), jnp.float32),
        ),
    )(x, W1, deg2)

    s1 = _agg16(hs, src, dst)

    gs = pl.pallas_call(
        _tcb_body,
        out_shape=jax.ShapeDtypeStruct((NP, DO), jnp.float32),
    )(s1, dis, b1, W2p)

    s2 = _agg4(gs, src, dst)

    out = pl.pallas_call(
        _tcc_body,
        out_shape=jax.ShapeDtypeStruct((NP, 2), jnp.float32),
    )(s2, dis, b2)

    return out[:N]


# speedup vs baseline: 1.0922x; 1.0922x over previous
import functools

import jax
import jax.numpy as jnp
from jax import lax
from jax.experimental import pallas as pl
from jax.experimental.pallas import tpu as pltpu
from jax.experimental.pallas import tpu_sc as plsc

N = 10000
D_IN = 128
DH = 16
DO = 8
NP = 10240
NZ = NP - N
E_RAW = 320000
E_TOT = E_RAW + N
NC, NS, L = 2, 16, 16
CHUNK = 128
NBUF = 4
CH0 = 104
CH1 = 60
CHMAX = max(CH0, CH1)
TOTAL_CH = NS * (CH0 + CH1)
ROWS_PAD = TOTAL_CH + CHMAX - min(CH0, CH1)
E_PAD = ROWS_PAD * CHUNK
RPS = NP // NS

_mesh = functools.partial(
    plsc.VectorSubcoreMesh, core_axis_name="c", subcore_axis_name="s"
)
_sc_params = pltpu.CompilerParams(use_tc_tiling_on_sc=False)


def _tile_plan(c, s):
    start = jnp.where(c == 0, s * CH0, NS * CH0 + s * CH1)
    nch = jnp.where(c == 0, CH0, CH1)
    ngrp = jnp.where(c == 0, CH0 // NBUF, CH1 // NBUF)
    return start, nch, ngrp


@functools.partial(
    pl.kernel,
    out_type=jax.ShapeDtypeStruct((NC, NP), jnp.float32),
    mesh=_mesh(),
    scratch_types=[
        pltpu.VMEM_SHARED((NP,), jnp.float32),
        pltpu.VMEM((CHMAX, CHUNK), jnp.int32),
        pltpu.VMEM((CHUNK,), jnp.float32),
        pltpu.VMEM((RPS,), jnp.float32),
        pltpu.SemaphoreType.DMA((NBUF,)),
        pltpu.SemaphoreType.DMA,
    ],
    compiler_params=_sc_params,
)
def _deg_kernel(dst_hbm, out_hbm, acc_sh, dst_v, ones_v, zero_v, ssem, isem):
    c = lax.axis_index("c")
    s = lax.axis_index("s")
    start, nch, ngrp = _tile_plan(c, s)

    idx_cp = pltpu.async_copy(dst_hbm.at[pl.ds(start, CHMAX)], dst_v, isem)

    def fill_ones(i, carry):
        ones_v[pl.ds(i * L, L)] = jnp.full((L,), 1.0, jnp.float32)
        return carry

    lax.fori_loop(0, CHUNK // L, fill_ones, 0)

    def fill_zero(i, carry):
        zero_v[pl.ds(i * L, L)] = jnp.zeros((L,), jnp.float32)
        return carry

    lax.fori_loop(0, RPS // L, fill_zero, 0)

    pltpu.sync_copy(zero_v, acc_sh.at[pl.ds(s * RPS, RPS)])
    idx_cp.wait()
    plsc.subcore_barrier()

    def group(g, carry):
        for b in range(NBUF):
            j = g * NBUF + b
            pltpu.async_copy(ones_v, acc_sh.at[dst_v.at[j]], ssem.at[b],
                             add=True)
        for b in range(NBUF):
            pltpu.make_async_copy(ones_v, acc_sh.at[dst_v.at[b]],
                                  ssem.at[b]).wait()
        return carry

    lax.fori_loop(0, ngrp, group, 0)

    plsc.subcore_barrier()
    pltpu.sync_copy(
        acc_sh.at[pl.ds(s * RPS, RPS)], out_hbm.at[c, pl.ds(s * RPS, RPS)]
    )


def _agg_body(D, feat_hbm, src_hbm, dst_hbm, out_hbm,
              acc_sh, src_v, dst_v, rows_v, gsem, ssem, isem):
    c = lax.axis_index("c")
    s = lax.axis_index("s")
    start, nch, ngrp = _tile_plan(c, s)

    cp_s = pltpu.async_copy(src_hbm.at[pl.ds(start, CHMAX)], src_v, isem)
    cp_d = pltpu.async_copy(dst_hbm.at[pl.ds(start, CHMAX)], dst_v, isem)

    base = s * RPS
    pltpu.sync_copy(feat_hbm.at[pl.ds(N, NZ)], acc_sh.at[pl.ds(base, NZ)])
    pltpu.sync_copy(feat_hbm.at[pl.ds(N, NZ)],
                    acc_sh.at[pl.ds(base + NZ, NZ)])
    pltpu.sync_copy(feat_hbm.at[pl.ds(N, RPS - 2 * NZ)],
                    acc_sh.at[pl.ds(base + 2 * NZ, RPS - 2 * NZ)])
    cp_s.wait()
    cp_d.wait()
    plsc.subcore_barrier()

    for b in range(NBUF):
        pltpu.async_copy(feat_hbm.at[src_v.at[b]], rows_v.at[b], gsem.at[b])

    def group(g, carry):
        for b in range(NBUF):
            j = g * NBUF + b
            pltpu.make_async_copy(feat_hbm.at[src_v.at[b]], rows_v.at[b],
                                  gsem.at[b]).wait()
            pltpu.async_copy(rows_v.at[b], acc_sh.at[dst_v.at[j]],
                             ssem.at[b], add=True)
        for b in range(NBUF):
            nxt = g * NBUF + b + NBUF
            pltpu.make_async_copy(rows_v.at[b], acc_sh.at[dst_v.at[b]],
                                  ssem.at[b]).wait()

            @pl.when(nxt < nch)
            def _():
                pltpu.async_copy(feat_hbm.at[src_v.at[nxt]], rows_v.at[b],
                                 gsem.at[b])

        return carry

    lax.fori_loop(0, ngrp, group, 0)

    plsc.subcore_barrier()
    pltpu.sync_copy(
        acc_sh.at[pl.ds(s * RPS, RPS)], out_hbm.at[c, pl.ds(s * RPS, RPS)]
    )


def _make_agg(D):
    return functools.partial(
        pl.kernel,
        out_type=jax.ShapeDtypeStruct((NC, NP, D), jnp.float32),
        mesh=_mesh(),
        scratch_types=[
            pltpu.VMEM_SHARED((NP, D), jnp.float32),
            pltpu.VMEM((CHMAX, CHUNK), jnp.int32),
            pltpu.VMEM((CHMAX, CHUNK), jnp.int32),
            pltpu.VMEM((NBUF, CHUNK, D), jnp.float32),
            pltpu.SemaphoreType.DMA((NBUF,)),
            pltpu.SemaphoreType.DMA((NBUF,)),
            pltpu.SemaphoreType.DMA,
        ],
        compiler_params=_sc_params,
    )(functools.partial(_agg_body, D))


_agg16 = _make_agg(DH)
_agg4 = _make_agg(DO)


def _tca_body(x_ref, w1_ref, deg_ref, hs_ref, dis_ref):
    deg = deg_ref[0] + deg_ref[1]
    dis = jnp.where(deg > 0, lax.rsqrt(jnp.maximum(deg, 1e-12)), 0.0)
    dis_ref[...] = dis
    h = jnp.dot(x_ref[...], w1_ref[...], preferred_element_type=jnp.float32)
    h = jnp.concatenate([h, jnp.zeros((NP - N, DH), jnp.float32)], axis=0)
    hs_ref[...] = h * dis[:, None]


def _tcb_body(s1_ref, dis_ref, b1_ref, w2_ref, gs_ref):
    s = s1_ref[0] + s1_ref[1]
    dis = dis_ref[...]
    x2 = jnp.maximum(s * dis[:, None] + b1_ref[...][None, :], 0.0)
    g = jnp.dot(x2, w2_ref[...], preferred_element_type=jnp.float32)
    g = g * dis[:, None]
    row = lax.broadcasted_iota(jnp.int32, (NP, DO), 0)
    gs_ref[...] = jnp.where(row < N, g, 0.0)


def _tcc_body(s2_ref, dis_ref, b2_ref, out_ref):
    su = s2_ref[0] + s2_ref[1]
    dis = dis_ref[...]
    z = su[:, :2] * dis[:, None] + b2_ref[...][None, :]
    m = jnp.max(z, axis=1, keepdims=True)
    lse = m + jnp.log(jnp.sum(jnp.exp(z - m), axis=1, keepdims=True))
    out_ref[...] = z - lse


def kernel(x, edge_index, W1, b1, W2, b2):
    loop = jnp.arange(N, dtype=jnp.int32)
    pad = jnp.full((E_PAD - E_TOT,), N, jnp.int32)
    src = jnp.concatenate([edge_index[0], loop, pad]).reshape(ROWS_PAD, CHUNK)
    dst = jnp.concatenate([edge_index[1], loop, pad]).reshape(ROWS_PAD, CHUNK)
    W2p = jnp.pad(W2, ((0, 0), (0, DO - 2)))

    deg2 = _deg_kernel(dst)

    hs, dis = pl.pallas_call(
        _tca_body,
        out_shape=(
            jax.ShapeDtypeStruct((NP, DH), jnp.float32),
            jax.ShapeDtypeStruct((NP,), jnp.float32),
        ),
    )(x, W1, deg2)

    s1 = _agg16(hs, src, dst)

    gs = pl.pallas_call(
        _tcb_body,
        out_shape=jax.ShapeDtypeStruct((NP, DO), jnp.float32),
    )(s1, dis, b1, W2p)

    s2 = _agg4(gs, src, dst)

    out = pl.pallas_call(
        _tcc_body,
        out_shape=jax.ShapeDtypeStruct((NP, 2), jnp.float32),
    )(s2, dis, b2)

    return out[:N]


# speedup vs baseline: 1.1235x; 1.0286x over previous
import functools

import jax
import jax.numpy as jnp
from jax import lax
from jax.experimental import pallas as pl
from jax.experimental.pallas import tpu as pltpu
from jax.experimental.pallas import tpu_sc as plsc

N = 10000
D_IN = 128
DH = 16
DO = 8
NP = 10240
NZ = NP - N
E_RAW = 320000
E_TOT = E_RAW + N
NC, NS, L = 2, 16, 16
CHUNK = 128
NBUF = 4
CH_SUM = 164
CHMAX = 104
TOTAL_CH = NS * CH_SUM
ROWS_PAD = TOTAL_CH + 44
E_PAD = ROWS_PAD * CHUNK
RPS = NP // NS

CH_DEG = (100, 64)
CH_AGG1 = (92, 72)
CH_AGG2 = (100, 64)

_mesh = functools.partial(
    plsc.VectorSubcoreMesh, core_axis_name="c", subcore_axis_name="s"
)
_sc_params = pltpu.CompilerParams(use_tc_tiling_on_sc=False)


def _tile_plan(split, c, s):
    ch0, ch1 = split
    assert ch0 + ch1 == CH_SUM and ch0 % NBUF == 0 and ch1 % NBUF == 0
    assert ch0 <= CHMAX and ch0 - ch1 <= ROWS_PAD - TOTAL_CH
    start = jnp.where(c == 0, s * ch0, NS * ch0 + s * ch1)
    nch = jnp.where(c == 0, ch0, ch1)
    ngrp = jnp.where(c == 0, ch0 // NBUF, ch1 // NBUF)
    return start, nch, ngrp


@functools.partial(
    pl.kernel,
    out_type=jax.ShapeDtypeStruct((NC, NP), jnp.float32),
    mesh=_mesh(),
    scratch_types=[
        pltpu.VMEM_SHARED((NP,), jnp.float32),
        pltpu.VMEM((CHMAX, CHUNK), jnp.int32),
        pltpu.VMEM((CHUNK,), jnp.float32),
        pltpu.VMEM((RPS,), jnp.float32),
        pltpu.SemaphoreType.DMA((NBUF,)),
        pltpu.SemaphoreType.DMA,
    ],
    compiler_params=_sc_params,
)
def _deg_kernel(dst_hbm, out_hbm, acc_sh, dst_v, ones_v, zero_v, ssem, isem):
    c = lax.axis_index("c")
    s = lax.axis_index("s")
    start, nch, ngrp = _tile_plan(CH_DEG, c, s)

    idx_cp = pltpu.async_copy(dst_hbm.at[pl.ds(start, CHMAX)], dst_v, isem)

    def fill_ones(i, carry):
        ones_v[pl.ds(i * L, L)] = jnp.full((L,), 1.0, jnp.float32)
        return carry

    lax.fori_loop(0, CHUNK // L, fill_ones, 0)

    def fill_zero(i, carry):
        zero_v[pl.ds(i * L, L)] = jnp.zeros((L,), jnp.float32)
        return carry

    lax.fori_loop(0, RPS // L, fill_zero, 0)

    pltpu.sync_copy(zero_v, acc_sh.at[pl.ds(s * RPS, RPS)])
    idx_cp.wait()
    plsc.subcore_barrier()

    def group(g, carry):
        for b in range(NBUF):
            j = g * NBUF + b
            pltpu.async_copy(ones_v, acc_sh.at[dst_v.at[j]], ssem.at[b],
                             add=True)
        for b in range(NBUF):
            pltpu.make_async_copy(ones_v, acc_sh.at[dst_v.at[b]],
                                  ssem.at[b]).wait()
        return carry

    lax.fori_loop(0, ngrp, group, 0)

    plsc.subcore_barrier()
    pltpu.sync_copy(
        acc_sh.at[pl.ds(s * RPS, RPS)], out_hbm.at[c, pl.ds(s * RPS, RPS)]
    )


def _agg_body(D, split, feat_hbm, src_hbm, dst_hbm, out_hbm,
              acc_sh, src_v, dst_v, rows_v, gsem, ssem, isem):
    c = lax.axis_index("c")
    s = lax.axis_index("s")
    start, nch, ngrp = _tile_plan(split, c, s)

    cp_s = pltpu.async_copy(src_hbm.at[pl.ds(start, CHMAX)], src_v, isem)
    cp_d = pltpu.async_copy(dst_hbm.at[pl.ds(start, CHMAX)], dst_v, isem)

    base = s * RPS
    pltpu.sync_copy(feat_hbm.at[pl.ds(N, NZ)], acc_sh.at[pl.ds(base, NZ)])
    pltpu.sync_copy(feat_hbm.at[pl.ds(N, NZ)],
                    acc_sh.at[pl.ds(base + NZ, NZ)])
    pltpu.sync_copy(feat_hbm.at[pl.ds(N, RPS - 2 * NZ)],
                    acc_sh.at[pl.ds(base + 2 * NZ, RPS - 2 * NZ)])
    cp_s.wait()
    cp_d.wait()
    plsc.subcore_barrier()

    for b in range(NBUF):
        pltpu.async_copy(feat_hbm.at[src_v.at[b]], rows_v.at[b], gsem.at[b])

    def group(g, carry):
        for b in range(NBUF):
            j = g * NBUF + b
            pltpu.make_async_copy(feat_hbm.at[src_v.at[b]], rows_v.at[b],
                                  gsem.at[b]).wait()
            pltpu.async_copy(rows_v.at[b], acc_sh.at[dst_v.at[j]],
                             ssem.at[b], add=True)
        for b in range(NBUF):
            nxt = g * NBUF + b + NBUF
            pltpu.make_async_copy(rows_v.at[b], acc_sh.at[dst_v.at[b]],
                                  ssem.at[b]).wait()

            @pl.when(nxt < nch)
            def _():
                pltpu.async_copy(feat_hbm.at[src_v.at[nxt]], rows_v.at[b],
                                 gsem.at[b])

        return carry

    lax.fori_loop(0, ngrp, group, 0)

    plsc.subcore_barrier()
    pltpu.sync_copy(
        acc_sh.at[pl.ds(s * RPS, RPS)], out_hbm.at[c, pl.ds(s * RPS, RPS)]
    )


def _make_agg(D, split):
    return functools.partial(
        pl.kernel,
        out_type=jax.ShapeDtypeStruct((NC, NP, D), jnp.float32),
        mesh=_mesh(),
        scratch_types=[
            pltpu.VMEM_SHARED((NP, D), jnp.float32),
            pltpu.VMEM((CHMAX, CHUNK), jnp.int32),
            pltpu.VMEM((CHMAX, CHUNK), jnp.int32),
            pltpu.VMEM((NBUF, CHUNK, D), jnp.float32),
            pltpu.SemaphoreType.DMA((NBUF,)),
            pltpu.SemaphoreType.DMA((NBUF,)),
            pltpu.SemaphoreType.DMA,
        ],
        compiler_params=_sc_params,
    )(functools.partial(_agg_body, D, split))


_agg16 = _make_agg(DH, CH_AGG1)
_agg8 = _make_agg(DO, CH_AGG2)


def _tca0_body(x_ref, w1_ref, h_ref):
    h_ref[...] = jnp.dot(x_ref[...], w1_ref[...],
                         preferred_element_type=jnp.float32)


def _tca1_body(h_ref, deg_ref, hs_ref, dis_ref):
    deg = deg_ref[0] + deg_ref[1]
    dis = jnp.where(deg > 0, lax.rsqrt(jnp.maximum(deg, 1e-12)), 0.0)
    dis_ref[...] = dis
    h = jnp.concatenate([h_ref[...], jnp.zeros((NP - N, DH), jnp.float32)],
                        axis=0)
    hs_ref[...] = h * dis[:, None]


def _tcb_body(s1_ref, dis_ref, b1_ref, w2_ref, gs_ref):
    s = s1_ref[0] + s1_ref[1]
    dis = dis_ref[...]
    x2 = jnp.maximum(s * dis[:, None] + b1_ref[...][None, :], 0.0)
    g = jnp.dot(x2, w2_ref[...], preferred_element_type=jnp.float32)
    g = g * dis[:, None]
    row = lax.broadcasted_iota(jnp.int32, (NP, DO), 0)
    gs_ref[...] = jnp.where(row < N, g, 0.0)


def _tcc_body(s2_ref, dis_ref, b2_ref, out_ref):
    su = s2_ref[0] + s2_ref[1]
    dis = dis_ref[...]
    z = su[:, :2] * dis[:, None] + b2_ref[...][None, :]
    z = z[:N]
    m = jnp.max(z, axis=1, keepdims=True)
    lse = m + jnp.log(jnp.sum(jnp.exp(z - m), axis=1, keepdims=True))
    out_ref[...] = z - lse


def kernel(x, edge_index, W1, b1, W2, b2):
    loop = jnp.arange(N, dtype=jnp.int32)
    pad = jnp.full((E_PAD - E_TOT,), N, jnp.int32)
    src = jnp.concatenate([edge_index[0], loop, pad]).reshape(ROWS_PAD, CHUNK)
    dst = jnp.concatenate([edge_index[1], loop, pad]).reshape(ROWS_PAD, CHUNK)
    W2p = jnp.pad(W2, ((0, 0), (0, DO - 2)))

    deg2 = _deg_kernel(dst)

    h = pl.pallas_call(
        _tca0_body,
        out_shape=jax.ShapeDtypeStruct((N, DH), jnp.float32),
    )(x, W1)

    hs, dis = pl.pallas_call(
        _tca1_body,
        out_shape=(
            jax.ShapeDtypeStruct((NP, DH), jnp.float32),
            jax.ShapeDtypeStruct((NP,), jnp.float32),
        ),
    )(h, deg2)

    s1 = _agg16(hs, src, dst)

    gs = pl.pallas_call(
        _tcb_body,
        out_shape=jax.ShapeDtypeStruct((NP, DO), jnp.float32),
    )(s1, dis, b1, W2p)

    s2 = _agg8(gs, src, dst)

    out = pl.pallas_call(
        _tcc_body,
        out_shape=jax.ShapeDtypeStruct((N, 2), jnp.float32),
    )(s2, dis, b2)

    return out
